# Initial kernel scaffold; baseline (speedup 1.0000x reference)
import jax
import jax.numpy as jnp
from jax.experimental import pallas as pl


def kernel(de1, de2, f, W1, W2):
    raise NotImplementedError("write your pallas kernel here")


# speedup vs baseline: 2.6547x; 2.6547x over previous
import jax
import jax.numpy as jnp
from jax import lax
from jax.experimental import pallas as pl
from jax.experimental.pallas import tpu as pltpu
from jax.experimental.pallas import tpu_sc as plsc

NUM_EMB = 1000
EMB = 32
B = 16384
L = 200
N = B * L
OUT_W = 2 * EMB + 1

NC = 2
NS = 16
NW = NC * NS
PER_W = N // NW
CHUNK = 128
STEPS = PER_W // CHUNK
GRP = CHUNK // 16


def _body(de1_hbm, de2_hbm, f_hbm, w1_hbm, w2_hbm, out_hbm,
          w1_v, w2_v, idx1_v, idx2_v, f_v, out_v, sem):
    wid = lax.axis_index("s") * NC + lax.axis_index("c")
    base0 = wid * PER_W

    pltpu.sync_copy(w1_hbm, w1_v)
    pltpu.sync_copy(w2_hbm, w2_v)

    lanes = lax.iota(jnp.int32, 16)

    def step(i, carry):
        base = base0 + i * CHUNK
        pltpu.sync_copy(de1_hbm.at[pl.ds(base, CHUNK)], idx1_v)
        pltpu.sync_copy(de2_hbm.at[pl.ds(base, CHUNK)], idx2_v)
        pltpu.sync_copy(f_hbm.at[pl.ds(base, CHUNK)], f_v)

        def group(j, carry2):
            idx1 = idx1_v[pl.ds(j * 16, 16)]
            idx2 = idx2_v[pl.ds(j * 16, 16)]
            src1 = idx1 * EMB
            src2 = idx2 * EMB
            dst0 = (lanes + j * 16) * OUT_W
            for c in range(EMB):
                va = plsc.load_gather(w1_v, [src1 + c])
                plsc.store_scatter(out_v, [dst0 + c], va)
                vb = plsc.load_gather(w2_v, [src2 + c])
                plsc.store_scatter(out_v, [dst0 + (EMB + c)], vb)
            fv = f_v[pl.ds(j * 16, 16)]
            plsc.store_scatter(out_v, [dst0 + 2 * EMB], fv)
            return carry2

        lax.fori_loop(0, GRP, group, 0)
        pltpu.sync_copy(out_v, out_hbm.at[pl.ds(base * OUT_W, CHUNK * OUT_W)])
        return carry

    lax.fori_loop(0, STEPS, step, 0)


@jax.jit
def _run(de1f, de2f, ff, W1f, W2f):
    mesh = plsc.VectorSubcoreMesh(core_axis_name="c", subcore_axis_name="s")
    return pl.kernel(
        _body,
        out_type=jax.ShapeDtypeStruct((N * OUT_W,), jnp.float32),
        mesh=mesh,
        scratch_types=[
            pltpu.VMEM((NUM_EMB * EMB,), jnp.float32),
            pltpu.VMEM((NUM_EMB * EMB,), jnp.float32),
            pltpu.VMEM((CHUNK,), jnp.int32),
            pltpu.VMEM((CHUNK,), jnp.int32),
            pltpu.VMEM((CHUNK,), jnp.float32),
            pltpu.VMEM((CHUNK * OUT_W,), jnp.float32),
            pltpu.SemaphoreType.DMA,
        ],
        compiler_params=pltpu.CompilerParams(
            use_tc_tiling_on_sc=False, needs_layout_passes=False),
    )(de1f, de2f, ff, W1f, W2f)


def kernel(de1, de2, f, W1, W2):
    out = _run(de1.reshape(N), de2.reshape(N), f.reshape(N),
               W1.reshape(NUM_EMB * EMB), W2.reshape(NUM_EMB * EMB))
    return out.reshape(B, L, OUT_W)


# speedup vs baseline: 3.1151x; 1.1734x over previous
import jax
import jax.numpy as jnp
from jax import lax
from jax.experimental import pallas as pl
from jax.experimental.pallas import tpu as pltpu
from jax.experimental.pallas import tpu_sc as plsc

NUM_EMB = 1000
EMB = 32
B = 16384
L = 200
N = B * L
OUT_W = 2 * EMB + 1

NC = 2
NS = 16
NW = NC * NS
PER_W = N // NW
CHUNK = 256
STEPS = PER_W // CHUNK
GRP = CHUNK // 16
CW = CHUNK * OUT_W


def _body(de1_hbm, de2_hbm, f_hbm, w1_hbm, w2_hbm, out_hbm,
          w1_v, w2_v, idx1_v, idx2_v, f_v, out_v, in_sems, out_sems):
    wid = lax.axis_index("s") * NC + lax.axis_index("c")
    base0 = wid * PER_W

    pltpu.sync_copy(w1_hbm, w1_v)
    pltpu.sync_copy(w2_hbm, w2_v)

    lanes = lax.iota(jnp.int32, 16)

    def start_in(i, b):
        base = base0 + i * CHUNK
        pltpu.async_copy(de1_hbm.at[pl.ds(base, CHUNK)], idx1_v[b], in_sems[b])
        pltpu.async_copy(de2_hbm.at[pl.ds(base, CHUNK)], idx2_v[b], in_sems[b])
        pltpu.async_copy(f_hbm.at[pl.ds(base, CHUNK)], f_v[b], in_sems[b])

    def wait_in(b):
        pltpu.make_async_copy(de1_hbm.at[pl.ds(0, CHUNK)], idx1_v[b],
                              in_sems[b]).wait()
        pltpu.make_async_copy(de2_hbm.at[pl.ds(0, CHUNK)], idx2_v[b],
                              in_sems[b]).wait()
        pltpu.make_async_copy(f_hbm.at[pl.ds(0, CHUNK)], f_v[b],
                              in_sems[b]).wait()

    def compute(b):
        def group(j, carry2):
            idx1 = idx1_v[b][pl.ds(j * 16, 16)]
            idx2 = idx2_v[b][pl.ds(j * 16, 16)]
            src1 = idx1 * EMB
            src2 = idx2 * EMB
            dst0 = (lanes + j * 16) * OUT_W
            for c in range(EMB):
                va = plsc.load_gather(w1_v, [src1 + c])
                plsc.store_scatter(out_v[b], [dst0 + c], va)
                vb = plsc.load_gather(w2_v, [src2 + c])
                plsc.store_scatter(out_v[b], [dst0 + (EMB + c)], vb)
            fv = f_v[b][pl.ds(j * 16, 16)]
            plsc.store_scatter(out_v[b], [dst0 + 2 * EMB], fv)
            return carry2

        lax.fori_loop(0, GRP, group, 0)

    def start_out(i, b):
        base = base0 + i * CHUNK
        pltpu.async_copy(out_v[b], out_hbm.at[pl.ds(base * OUT_W, CW)],
                         out_sems[b])

    def wait_out(b):
        pltpu.make_async_copy(out_v[b], out_hbm.at[pl.ds(0, CW)],
                              out_sems[b]).wait()

    start_in(0, 0)

    def step(k, carry):
        i0 = 2 * k
        start_in(i0 + 1, 1)
        wait_in(0)

        @pl.when(k > 0)
        def _():
            wait_out(0)

        compute(0)
        start_out(i0, 0)

        @pl.when(k < STEPS // 2 - 1)
        def _():
            start_in(i0 + 2, 0)

        wait_in(1)

        @pl.when(k > 0)
        def _():
            wait_out(1)

        compute(1)
        start_out(i0 + 1, 1)
        return carry

    lax.fori_loop(0, STEPS // 2, step, 0)
    wait_out(0)
    wait_out(1)


@jax.jit
def _run(de1f, de2f, ff, W1f, W2f):
    mesh = plsc.VectorSubcoreMesh(core_axis_name="c", subcore_axis_name="s")
    return pl.kernel(
        _body,
        out_type=jax.ShapeDtypeStruct((N * OUT_W,), jnp.float32),
        mesh=mesh,
        scratch_types=[
            pltpu.VMEM((NUM_EMB * EMB,), jnp.float32),
            pltpu.VMEM((NUM_EMB * EMB,), jnp.float32),
            [pltpu.VMEM((CHUNK,), jnp.int32) for _ in range(2)],
            [pltpu.VMEM((CHUNK,), jnp.int32) for _ in range(2)],
            [pltpu.VMEM((CHUNK,), jnp.float32) for _ in range(2)],
            [pltpu.VMEM((CW,), jnp.float32) for _ in range(2)],
            [pltpu.SemaphoreType.DMA for _ in range(2)],
            [pltpu.SemaphoreType.DMA for _ in range(2)],
        ],
        compiler_params=pltpu.CompilerParams(
            use_tc_tiling_on_sc=False, needs_layout_passes=False),
    )(de1f, de2f, ff, W1f, W2f)


def kernel(de1, de2, f, W1, W2):
    out = _run(de1.reshape(N), de2.reshape(N), f.reshape(N),
               W1.reshape(NUM_EMB * EMB), W2.reshape(NUM_EMB * EMB))
    return out.reshape(B, L, OUT_W)


# speedup vs baseline: 3.4522x; 1.1082x over previous
import jax
import jax.numpy as jnp
from jax import lax
from jax.experimental import pallas as pl
from jax.experimental.pallas import tpu as pltpu
from jax.experimental.pallas import tpu_sc as plsc

NUM_EMB = 1000
EMB = 32
B = 16384
L = 200
N = B * L
OUT_W = 2 * EMB + 1

NC = 2
NS = 16
NW = NC * NS
PER_W = N // NW
CHUNK = 256
STEPS = PER_W // CHUNK
GRP = CHUNK // 16
CW = CHUNK * OUT_W


def _body(de1_hbm, de2_hbm, f_hbm, w1_hbm, w2_hbm, out_hbm,
          w1_v, w2_v, idx1_v, idx2_v, f_v, out_v, in_sems, out_sems):
    wid = lax.axis_index("s") * NC + lax.axis_index("c")
    base0 = wid * PER_W

    pltpu.sync_copy(w1_hbm, w1_v)
    pltpu.sync_copy(w2_hbm, w2_v)

    lanes = lax.iota(jnp.int32, 16)

    def start_in(i, b):
        base = base0 + i * CHUNK
        pltpu.async_copy(de1_hbm.at[pl.ds(base, CHUNK)], idx1_v[b], in_sems[b])
        pltpu.async_copy(de2_hbm.at[pl.ds(base, CHUNK)], idx2_v[b], in_sems[b])
        pltpu.async_copy(f_hbm.at[pl.ds(base, CHUNK)], f_v[b], in_sems[b])

    def wait_in(b):
        pltpu.make_async_copy(de1_hbm.at[pl.ds(0, CHUNK)], idx1_v[b],
                              in_sems[b]).wait()
        pltpu.make_async_copy(de2_hbm.at[pl.ds(0, CHUNK)], idx2_v[b],
                              in_sems[b]).wait()
        pltpu.make_async_copy(f_hbm.at[pl.ds(0, CHUNK)], f_v[b],
                              in_sems[b]).wait()

    def compute(b):
        @plsc.parallel_loop(0, GRP, unroll=2)
        def group(j):
            idx1 = idx1_v[b][pl.ds(j * 16, 16)]
            idx2 = idx2_v[b][pl.ds(j * 16, 16)]
            src1 = idx1 * EMB
            src2 = idx2 * EMB
            dst0 = (lanes + j * 16) * OUT_W
            for c in range(EMB):
                va = plsc.load_gather(w1_v, [src1 + c])
                plsc.store_scatter(out_v[b], [dst0 + c], va)
                vb = plsc.load_gather(w2_v, [src2 + c])
                plsc.store_scatter(out_v[b], [dst0 + (EMB + c)], vb)
            fv = f_v[b][pl.ds(j * 16, 16)]
            plsc.store_scatter(out_v[b], [dst0 + 2 * EMB], fv)

    def start_out(i, b):
        base = base0 + i * CHUNK
        pltpu.async_copy(out_v[b], out_hbm.at[pl.ds(base * OUT_W, CW)],
                         out_sems[b])

    def wait_out(b):
        pltpu.make_async_copy(out_v[b], out_hbm.at[pl.ds(0, CW)],
                              out_sems[b]).wait()

    start_in(0, 0)

    def step(k, carry):
        i0 = 2 * k
        start_in(i0 + 1, 1)
        wait_in(0)

        @pl.when(k > 0)
        def _():
            wait_out(0)

        compute(0)
        start_out(i0, 0)

        @pl.when(k < STEPS // 2 - 1)
        def _():
            start_in(i0 + 2, 0)

        wait_in(1)

        @pl.when(k > 0)
        def _():
            wait_out(1)

        compute(1)
        start_out(i0 + 1, 1)
        return carry

    lax.fori_loop(0, STEPS // 2, step, 0)
    wait_out(0)
    wait_out(1)


@jax.jit
def _run(de1f, de2f, ff, W1f, W2f):
    mesh = plsc.VectorSubcoreMesh(core_axis_name="c", subcore_axis_name="s")
    return pl.kernel(
        _body,
        out_type=jax.ShapeDtypeStruct((N * OUT_W,), jnp.float32),
        mesh=mesh,
        scratch_types=[
            pltpu.VMEM((NUM_EMB * EMB,), jnp.float32),
            pltpu.VMEM((NUM_EMB * EMB,), jnp.float32),
            [pltpu.VMEM((CHUNK,), jnp.int32) for _ in range(2)],
            [pltpu.VMEM((CHUNK,), jnp.int32) for _ in range(2)],
            [pltpu.VMEM((CHUNK,), jnp.float32) for _ in range(2)],
            [pltpu.VMEM((CW,), jnp.float32) for _ in range(2)],
            [pltpu.SemaphoreType.DMA for _ in range(2)],
            [pltpu.SemaphoreType.DMA for _ in range(2)],
        ],
        compiler_params=pltpu.CompilerParams(
            use_tc_tiling_on_sc=False, needs_layout_passes=False),
    )(de1f, de2f, ff, W1f, W2f)


def kernel(de1, de2, f, W1, W2):
    out = _run(de1.reshape(N), de2.reshape(N), f.reshape(N),
               W1.reshape(NUM_EMB * EMB), W2.reshape(NUM_EMB * EMB))
    return out.reshape(B, L, OUT_W)


# speedup vs baseline: 3.4560x; 1.0011x over previous
import jax
import jax.numpy as jnp
from jax import lax
from jax.experimental import pallas as pl
from jax.experimental.pallas import tpu as pltpu
from jax.experimental.pallas import tpu_sc as plsc

NUM_EMB = 1000
EMB = 32
B = 16384
L = 200
N = B * L
OUT_W = 2 * EMB + 1

NC = 2
NS = 16
NW = NC * NS
PER_W = N // NW
CHUNK = 256
STEPS = PER_W // CHUNK
GRP = CHUNK // 16
CW = CHUNK * OUT_W


def _body(de1_hbm, de2_hbm, f_hbm, w1_hbm, w2_hbm, out_hbm,
          w1_v, w2_v, idx1_v, idx2_v, f_v, out_v, in_sems, out_sems):
    wid = lax.axis_index("s") * NC + lax.axis_index("c")
    base0 = wid * PER_W

    pltpu.sync_copy(w1_hbm, w1_v)
    pltpu.sync_copy(w2_hbm, w2_v)

    lanes = lax.iota(jnp.int32, 16)

    def start_in(i, b):
        base = base0 + i * CHUNK
        pltpu.async_copy(de1_hbm.at[pl.ds(base, CHUNK)], idx1_v[b], in_sems[b])
        pltpu.async_copy(de2_hbm.at[pl.ds(base, CHUNK)], idx2_v[b], in_sems[b])
        pltpu.async_copy(f_hbm.at[pl.ds(base, CHUNK)], f_v[b], in_sems[b])

    def wait_in(b):
        pltpu.make_async_copy(de1_hbm.at[pl.ds(0, CHUNK)], idx1_v[b],
                              in_sems[b]).wait()
        pltpu.make_async_copy(de2_hbm.at[pl.ds(0, CHUNK)], idx2_v[b],
                              in_sems[b]).wait()
        pltpu.make_async_copy(f_hbm.at[pl.ds(0, CHUNK)], f_v[b],
                              in_sems[b]).wait()

    def compute(b):
        @plsc.parallel_loop(0, GRP, unroll=2)
        def group(j):
            idx1 = idx1_v[b][pl.ds(j * 16, 16)]
            idx2 = idx2_v[b][pl.ds(j * 16, 16)]
            src1 = idx1 * EMB
            src2 = idx2 * EMB
            dst0 = (lanes + j * 16) * OUT_W
            for c in range(EMB):
                va = plsc.load_gather(w1_v, [src1 + c])
                plsc.store_scatter(out_v[b], [dst0 + c], va)
                vb = plsc.load_gather(w2_v, [src2 + c])
                plsc.store_scatter(out_v[b], [dst0 + (EMB + c)], vb)
            fv = f_v[b][pl.ds(j * 16, 16)]
            plsc.store_scatter(out_v[b], [dst0 + 2 * EMB], fv)

    def start_out(i, b):
        base = base0 + i * CHUNK
        pltpu.async_copy(out_v[b], out_hbm.at[pl.ds(base * OUT_W, CW)],
                         out_sems[b])

    def wait_out(b):
        pltpu.make_async_copy(out_v[b], out_hbm.at[pl.ds(0, CW)],
                              out_sems[b]).wait()

    start_in(0, 0)

    def step(k, carry):
        i0 = 2 * k
        start_in(i0 + 1, 1)
        wait_in(0)

        @pl.when(k > 0)
        def _():
            wait_out(0)

        compute(0)
        start_out(i0, 0)

        @pl.when(k < STEPS // 2 - 1)
        def _():
            start_in(i0 + 2, 0)

        wait_in(1)

        @pl.when(k > 0)
        def _():
            wait_out(1)

        compute(1)
        start_out(i0 + 1, 1)
        return carry

    lax.fori_loop(0, STEPS // 2, step, 0)
    wait_out(0)
    wait_out(1)


@jax.jit
def _run(de1f, de2f, ff, W1f, W2f):
    mesh = plsc.VectorSubcoreMesh(core_axis_name="c", subcore_axis_name="s")
    return pl.kernel(
        _body,
        out_type=jax.ShapeDtypeStruct((N * OUT_W,), jnp.float32),
        mesh=mesh,
        scratch_types=[
            pltpu.VMEM((NUM_EMB * EMB,), jnp.float32),
            pltpu.VMEM((NUM_EMB * EMB,), jnp.float32),
            [pltpu.VMEM((CHUNK,), jnp.int32) for _ in range(2)],
            [pltpu.VMEM((CHUNK,), jnp.int32) for _ in range(2)],
            [pltpu.VMEM((CHUNK,), jnp.float32) for _ in range(2)],
            [pltpu.VMEM((CW,), jnp.float32) for _ in range(2)],
            [pltpu.SemaphoreType.DMA for _ in range(2)],
            [pltpu.SemaphoreType.DMA for _ in range(2)],
        ],
        compiler_params=pltpu.CompilerParams(
            use_tc_tiling_on_sc=False, needs_layout_passes=False,
            disable_bounds_checks=True),
    )(de1f, de2f, ff, W1f, W2f)


def kernel(de1, de2, f, W1, W2):
    out = _run(de1.reshape(N), de2.reshape(N), f.reshape(N),
               W1.reshape(NUM_EMB * EMB), W2.reshape(NUM_EMB * EMB))
    return out.reshape(B, L, OUT_W)


# speedup vs baseline: 4.9784x; 1.4405x over previous
import jax
import jax.numpy as jnp
from jax import lax
from jax.experimental import pallas as pl
from jax.experimental.pallas import tpu as pltpu
from jax.experimental.pallas import tpu_sc as plsc

NUM_EMB = 1000
EMB = 32
B = 16384
L = 200
N = B * L
OUT_W = 2 * EMB + 1

NC = 2
NS = 16
NW = NC * NS
PER_W = N // NW
CHUNK = 256
STEPS = PER_W // CHUNK
GRP = CHUNK // 16
CW = CHUNK * OUT_W
PEMB = EMB + 1


def _body(de1_hbm, de2_hbm, f_hbm, w1_hbm, w2_hbm, out_hbm,
          w1_v, w2_v, idx1_v, idx2_v, f_v, out_v, in_sems, out_sems):
    wid = lax.axis_index("s") * NC + lax.axis_index("c")
    base0 = wid * PER_W

    pltpu.sync_copy(w1_hbm, w1_v)
    pltpu.sync_copy(w2_hbm, w2_v)

    lanes = lax.iota(jnp.int32, 16)

    def start_in(i, b):
        base = base0 + i * CHUNK
        pltpu.async_copy(de1_hbm.at[pl.ds(base, CHUNK)], idx1_v[b], in_sems[b])
        pltpu.async_copy(de2_hbm.at[pl.ds(base, CHUNK)], idx2_v[b], in_sems[b])
        pltpu.async_copy(f_hbm.at[pl.ds(base, CHUNK)], f_v[b], in_sems[b])

    def wait_in(b):
        pltpu.make_async_copy(de1_hbm.at[pl.ds(0, CHUNK)], idx1_v[b],
                              in_sems[b]).wait()
        pltpu.make_async_copy(de2_hbm.at[pl.ds(0, CHUNK)], idx2_v[b],
                              in_sems[b]).wait()
        pltpu.make_async_copy(f_hbm.at[pl.ds(0, CHUNK)], f_v[b],
                              in_sems[b]).wait()

    def compute(b):
        @plsc.parallel_loop(0, GRP, unroll=2)
        def group(j):
            idx1 = idx1_v[b][pl.ds(j * 16, 16)]
            idx2 = idx2_v[b][pl.ds(j * 16, 16)]
            src1 = idx1 * PEMB
            src2 = idx2 * PEMB
            dst0 = (lanes + j * 16) * OUT_W
            for c in range(EMB):
                va = plsc.load_gather(w1_v, [src1 + c])
                plsc.store_scatter(out_v[b], [dst0 + c], va)
                vb = plsc.load_gather(w2_v, [src2 + c])
                plsc.store_scatter(out_v[b], [dst0 + (EMB + c)], vb)
            fv = f_v[b][pl.ds(j * 16, 16)]
            plsc.store_scatter(out_v[b], [dst0 + 2 * EMB], fv)

    def start_out(i, b):
        base = base0 + i * CHUNK
        pltpu.async_copy(out_v[b], out_hbm.at[pl.ds(base * OUT_W, CW)],
                         out_sems[b])

    def wait_out(b):
        pltpu.make_async_copy(out_v[b], out_hbm.at[pl.ds(0, CW)],
                              out_sems[b]).wait()

    start_in(0, 0)

    def step(k, carry):
        i0 = 2 * k
        start_in(i0 + 1, 1)
        wait_in(0)

        @pl.when(k > 0)
        def _():
            wait_out(0)

        compute(0)
        start_out(i0, 0)

        @pl.when(k < STEPS // 2 - 1)
        def _():
            start_in(i0 + 2, 0)

        wait_in(1)

        @pl.when(k > 0)
        def _():
            wait_out(1)

        compute(1)
        start_out(i0 + 1, 1)
        return carry

    lax.fori_loop(0, STEPS // 2, step, 0)
    wait_out(0)
    wait_out(1)


@jax.jit
def _run(de1f, de2f, ff, W1f, W2f):
    mesh = plsc.VectorSubcoreMesh(core_axis_name="c", subcore_axis_name="s")
    return pl.kernel(
        _body,
        out_type=jax.ShapeDtypeStruct((N * OUT_W,), jnp.float32),
        mesh=mesh,
        scratch_types=[
            pltpu.VMEM((NUM_EMB * PEMB,), jnp.float32),
            pltpu.VMEM((NUM_EMB * PEMB,), jnp.float32),
            [pltpu.VMEM((CHUNK,), jnp.int32) for _ in range(2)],
            [pltpu.VMEM((CHUNK,), jnp.int32) for _ in range(2)],
            [pltpu.VMEM((CHUNK,), jnp.float32) for _ in range(2)],
            [pltpu.VMEM((CW,), jnp.float32) for _ in range(2)],
            [pltpu.SemaphoreType.DMA for _ in range(2)],
            [pltpu.SemaphoreType.DMA for _ in range(2)],
        ],
        compiler_params=pltpu.CompilerParams(
            use_tc_tiling_on_sc=False, needs_layout_passes=False,
            disable_bounds_checks=True),
    )(de1f, de2f, ff, W1f, W2f)


def kernel(de1, de2, f, W1, W2):
    W1p = jnp.pad(W1, ((0, 0), (0, 1))).reshape(NUM_EMB * PEMB)
    W2p = jnp.pad(W2, ((0, 0), (0, 1))).reshape(NUM_EMB * PEMB)
    out = _run(de1.reshape(N), de2.reshape(N), f.reshape(N), W1p, W2p)
    return out.reshape(B, L, OUT_W)


# speedup vs baseline: 6.8539x; 1.3767x over previous
import jax
import jax.numpy as jnp
from jax import lax
from jax.experimental import pallas as pl
from jax.experimental.pallas import tpu as pltpu
from jax.experimental.pallas import tpu_sc as plsc

NUM_EMB = 1000
EMB = 32
B = 16384
L = 200
N = B * L
OUT_W = 2 * EMB + 1

NC = 2
NS = 16
NW = NC * NS
PER_W = N // NW
CHUNK = 256
STEPS = PER_W // CHUNK
GRP = CHUNK // 16
CW = CHUNK * OUT_W


def _body(de1_hbm, de2_hbm, f_hbm, w1_hbm, w2_hbm, out_hbm,
          w1_v, w2_v, idx1_v, idx2_v, f_v, out_v, in_sems, out_sems):
    wid = lax.axis_index("s") * NC + lax.axis_index("c")
    base0 = wid * PER_W

    pltpu.sync_copy(w1_hbm, w1_v)
    pltpu.sync_copy(w2_hbm, w2_v)

    lanes = lax.iota(jnp.int32, 16)

    def start_in(i, b):
        base = base0 + i * CHUNK
        pltpu.async_copy(de1_hbm.at[pl.ds(base, CHUNK)], idx1_v[b], in_sems[b])
        pltpu.async_copy(de2_hbm.at[pl.ds(base, CHUNK)], idx2_v[b], in_sems[b])
        pltpu.async_copy(f_hbm.at[pl.ds(base, CHUNK)], f_v[b], in_sems[b])

    def wait_in(b):
        pltpu.make_async_copy(de1_hbm.at[pl.ds(0, CHUNK)], idx1_v[b],
                              in_sems[b]).wait()
        pltpu.make_async_copy(de2_hbm.at[pl.ds(0, CHUNK)], idx2_v[b],
                              in_sems[b]).wait()
        pltpu.make_async_copy(f_hbm.at[pl.ds(0, CHUNK)], f_v[b],
                              in_sems[b]).wait()

    dnums = lax.GatherDimensionNumbers(
        offset_dims=(), collapsed_slice_dims=(0,), start_index_map=(0,))

    def splat_lane(vec, t):
        return lax.gather(vec, jnp.full((16, 1), t, jnp.int32), dnums, (1,),
                          mode=lax.GatherScatterMode.PROMISE_IN_BOUNDS)

    def compute(b):
        @plsc.parallel_loop(0, GRP, unroll=2)
        def group(j):
            idx1 = idx1_v[b][pl.ds(j * 16, 16)]
            idx2 = idx2_v[b][pl.ds(j * 16, 16)]
            src1 = idx1 * EMB
            src2 = idx2 * EMB
            for t in range(16):
                s1 = splat_lane(src1, t)
                s2 = splat_lane(src2, t)
                a0 = plsc.load_gather(w1_v, [s1 + lanes])
                a1 = plsc.load_gather(w1_v, [s1 + (lanes + 16)])
                b0 = plsc.load_gather(w2_v, [s2 + lanes])
                b1 = plsc.load_gather(w2_v, [s2 + (lanes + 16)])
                off = (j * 16 + t) * OUT_W
                out_v[b][pl.ds(off, 16)] = a0
                out_v[b][pl.ds(off + 16, 16)] = a1
                out_v[b][pl.ds(off + 32, 16)] = b0
                out_v[b][pl.ds(off + 48, 16)] = b1
            fv = f_v[b][pl.ds(j * 16, 16)]
            dstf = (lanes + j * 16) * OUT_W + 2 * EMB
            plsc.store_scatter(out_v[b], [dstf], fv)

    def start_out(i, b):
        base = base0 + i * CHUNK
        pltpu.async_copy(out_v[b], out_hbm.at[pl.ds(base * OUT_W, CW)],
                         out_sems[b])

    def wait_out(b):
        pltpu.make_async_copy(out_v[b], out_hbm.at[pl.ds(0, CW)],
                              out_sems[b]).wait()

    start_in(0, 0)

    def step(k, carry):
        i0 = 2 * k
        start_in(i0 + 1, 1)
        wait_in(0)

        @pl.when(k > 0)
        def _():
            wait_out(0)

        compute(0)
        start_out(i0, 0)

        @pl.when(k < STEPS // 2 - 1)
        def _():
            start_in(i0 + 2, 0)

        wait_in(1)

        @pl.when(k > 0)
        def _():
            wait_out(1)

        compute(1)
        start_out(i0 + 1, 1)
        return carry

    lax.fori_loop(0, STEPS // 2, step, 0)
    wait_out(0)
    wait_out(1)


@jax.jit
def _run(de1f, de2f, ff, W1f, W2f):
    mesh = plsc.VectorSubcoreMesh(core_axis_name="c", subcore_axis_name="s")
    return pl.kernel(
        _body,
        out_type=jax.ShapeDtypeStruct((N * OUT_W,), jnp.float32),
        mesh=mesh,
        scratch_types=[
            pltpu.VMEM((NUM_EMB * EMB,), jnp.float32),
            pltpu.VMEM((NUM_EMB * EMB,), jnp.float32),
            [pltpu.VMEM((CHUNK,), jnp.int32) for _ in range(2)],
            [pltpu.VMEM((CHUNK,), jnp.int32) for _ in range(2)],
            [pltpu.VMEM((CHUNK,), jnp.float32) for _ in range(2)],
            [pltpu.VMEM((CW,), jnp.float32) for _ in range(2)],
            [pltpu.SemaphoreType.DMA for _ in range(2)],
            [pltpu.SemaphoreType.DMA for _ in range(2)],
        ],
        compiler_params=pltpu.CompilerParams(
            use_tc_tiling_on_sc=False, needs_layout_passes=False,
            disable_bounds_checks=True),
    )(de1f, de2f, ff, W1f, W2f)


def kernel(de1, de2, f, W1, W2):
    out = _run(de1.reshape(N), de2.reshape(N), f.reshape(N),
               W1.reshape(NUM_EMB * EMB), W2.reshape(NUM_EMB * EMB))
    return out.reshape(B, L, OUT_W)


# speedup vs baseline: 15.4689x; 2.2569x over previous
import jax
import jax.numpy as jnp
from jax import lax
from jax.experimental import pallas as pl
from jax.experimental.pallas import tpu as pltpu
from jax.experimental.pallas import tpu_sc as plsc

NUM_EMB = 1000
EMB = 32
B = 16384
L = 200
N = B * L
OUT_W = 2 * EMB + 1

NC = 2
NS = 16
NW = NC * NS
PER_W = N // NW
CHUNK = 160
STEPS = PER_W // CHUNK
GRP = CHUNK // 16
TPC = CHUNK // 8
TR = N // 8


def _body(de1_hbm, de2_hbm, f_hbm, w1_hbm, w2_hbm, out_hbm,
          w1_v, w2_v, idx1_v, idx2_v, f_v, out_v, in_sems, out_sems):
    wid = lax.axis_index("s") * NC + lax.axis_index("c")
    base0 = wid * PER_W

    pltpu.sync_copy(w1_hbm, w1_v)
    pltpu.sync_copy(w2_hbm, w2_v)

    lanes = lax.iota(jnp.int32, 16)

    def start_in(i, b):
        base = base0 + i * CHUNK
        pltpu.async_copy(de1_hbm.at[pl.ds(base, CHUNK)], idx1_v[b], in_sems[b])
        pltpu.async_copy(de2_hbm.at[pl.ds(base, CHUNK)], idx2_v[b], in_sems[b])
        pltpu.async_copy(f_hbm.at[pl.ds(base, CHUNK)], f_v[b], in_sems[b])

    def wait_in(b):
        pltpu.make_async_copy(de1_hbm.at[pl.ds(0, CHUNK)], idx1_v[b],
                              in_sems[b]).wait()
        pltpu.make_async_copy(de2_hbm.at[pl.ds(0, CHUNK)], idx2_v[b],
                              in_sems[b]).wait()
        pltpu.make_async_copy(f_hbm.at[pl.ds(0, CHUNK)], f_v[b],
                              in_sems[b]).wait()

    dnums = lax.GatherDimensionNumbers(
        offset_dims=(), collapsed_slice_dims=(0,), start_index_map=(0,))

    def splat_lane(vec, t):
        return lax.gather(vec, jnp.full((16, 1), t, jnp.int32), dnums, (1,),
                          mode=lax.GatherScatterMode.PROMISE_IN_BOUNDS)

    def compute(b):
        @plsc.parallel_loop(0, GRP, unroll=2)
        def group(j):
            idx1 = idx1_v[b][pl.ds(j * 16, 16)]
            idx2 = idx2_v[b][pl.ds(j * 16, 16)]
            src1 = idx1 * EMB
            src2 = idx2 * EMB
            for t in range(16):
                s1 = splat_lane(src1, t)
                s2 = splat_lane(src2, t)
                a0 = plsc.load_gather(w1_v, [s1 + lanes])
                a1 = plsc.load_gather(w1_v, [s1 + (lanes + 16)])
                b0 = plsc.load_gather(w2_v, [s2 + lanes])
                b1 = plsc.load_gather(w2_v, [s2 + (lanes + 16)])
                tr = 2 * j + t // 8
                r = t % 8
                out_v[b][tr, r, pl.ds(0, 16)] = a0
                out_v[b][tr, r, pl.ds(16, 16)] = a1
                out_v[b][tr, r, pl.ds(32, 16)] = b0
                out_v[b][tr, r, pl.ds(48, 16)] = b1
            fv = f_v[b][pl.ds(j * 16, 16)]
            trv = 2 * j + lanes // 8
            rv = lanes % 8
            cv = jnp.full((16,), 2 * EMB, jnp.int32)
            plsc.store_scatter(out_v[b], [trv, rv, cv], fv)

    def start_out(i, b):
        base = base0 + i * CHUNK
        pltpu.async_copy(out_v[b], out_hbm.at[pl.ds(base // 8, TPC)],
                         out_sems[b])

    def wait_out(b):
        pltpu.make_async_copy(out_v[b], out_hbm.at[pl.ds(0, TPC)],
                              out_sems[b]).wait()

    start_in(0, 0)

    def step(k, carry):
        i0 = 2 * k
        start_in(i0 + 1, 1)
        wait_in(0)

        @pl.when(k > 0)
        def _():
            wait_out(0)

        compute(0)
        start_out(i0, 0)

        @pl.when(k < STEPS // 2 - 1)
        def _():
            start_in(i0 + 2, 0)

        wait_in(1)

        @pl.when(k > 0)
        def _():
            wait_out(1)

        compute(1)
        start_out(i0 + 1, 1)
        return carry

    lax.fori_loop(0, STEPS // 2, step, 0)
    wait_out(0)
    wait_out(1)


@jax.jit
def _run(de1f, de2f, ff, W1f, W2f):
    mesh = plsc.VectorSubcoreMesh(core_axis_name="c", subcore_axis_name="s")
    return pl.kernel(
        _body,
        out_type=jax.ShapeDtypeStruct((TR, 8, OUT_W), jnp.float32),
        mesh=mesh,
        scratch_types=[
            pltpu.VMEM((NUM_EMB * EMB,), jnp.float32),
            pltpu.VMEM((NUM_EMB * EMB,), jnp.float32),
            [pltpu.VMEM((CHUNK,), jnp.int32) for _ in range(2)],
            [pltpu.VMEM((CHUNK,), jnp.int32) for _ in range(2)],
            [pltpu.VMEM((CHUNK,), jnp.float32) for _ in range(2)],
            [pltpu.VMEM((TPC, 8, OUT_W), jnp.float32) for _ in range(2)],
            [pltpu.SemaphoreType.DMA for _ in range(2)],
            [pltpu.SemaphoreType.DMA for _ in range(2)],
        ],
        compiler_params=pltpu.CompilerParams(
            needs_layout_passes=False, disable_bounds_checks=True),
    )(de1f, de2f, ff, W1f, W2f)


def kernel(de1, de2, f, W1, W2):
    out = _run(de1.reshape(N), de2.reshape(N), f.reshape(N),
               W1.reshape(NUM_EMB * EMB), W2.reshape(NUM_EMB * EMB))
    return out.reshape(B, L, OUT_W)
